# Initial kernel scaffold; baseline (speedup 1.0000x reference)
import jax
import jax.numpy as jnp
from jax.experimental import pallas as pl


def kernel(Traj, cell):
    raise NotImplementedError("write your pallas kernel here")


# speedup vs baseline: 1.1322x; 1.1322x over previous
import numpy as np
import jax
import jax.numpy as jnp
from jax.experimental import pallas as pl
from jax.experimental.pallas import tpu as pltpu

_DR = 0.1
_LMAX = 6.0
_NBINS = 58
_NPAD = 64


def _rdf_body(tt_ref, diag_ref, invn_ref, out_ref):
    T = tt_ref.shape[0]
    hists = None
    for t in range(T):
        x = tt_ref[t, 0, :]
        y = tt_ref[t, 1, :]
        z = tt_ref[t, 2, :]
        lx = diag_ref[0, 0]
        ly = diag_ref[0, 1]
        lz = diag_ref[0, 2]
        dx = x[:, None] - x[None, :]
        dy = y[:, None] - y[None, :]
        dz = z[:, None] - z[None, :]
        dx = dx - jnp.floor(dx / lx + 0.5) * lx
        dy = dy - jnp.floor(dy / ly + 0.5) * ly
        dz = dz - jnp.floor(dz / lz + 0.5) * lz
        sq = dx * dx + dy * dy + dz * dz
        mask = (sq < _LMAX * _LMAX) & (sq != 0.0)
        u = jnp.where(mask, jnp.sqrt(sq) * (1.0 / _DR), 1e6)
        part = []
        for k in range(_NBINS):
            ck = k + 0.5
            arg = u - ck
            part.append(jnp.sum(jnp.exp(-0.5 * (arg * arg))))
        h = jnp.stack(part)
        hists = h if hists is None else hists + h
    h64 = jnp.concatenate([hists, jnp.zeros((_NPAD - _NBINS,), jnp.float32)])
    out_ref[0, :] = h64 * invn_ref[0, :]


def kernel(Traj, cell):
    T, natom, _ = Traj.shape
    tt = jnp.transpose(Traj, (0, 2, 1))
    diag = jnp.diag(cell).reshape(1, 3)
    det = jnp.linalg.det(cell)

    r_np = np.arange(0.5 * _DR, _LMAX - _DR * 2, _DR, dtype=np.float32)
    v = 4.0 * np.pi / 3.0 * ((r_np + 0.5 * _DR) ** 3 - (r_np - 0.5 * _DR) ** 3)
    base = 0.5 / np.sqrt(2.0 * np.pi) / T / v * 2.0 / ((natom - 1) * natom)
    invn = jnp.concatenate(
        [jnp.asarray(base, jnp.float32), jnp.zeros((_NPAD - _NBINS,), jnp.float32)]
    ).reshape(1, _NPAD) * det

    out = pl.pallas_call(
        _rdf_body,
        out_shape=jax.ShapeDtypeStruct((1, _NPAD), jnp.float32),
    )(tt, diag, invn)

    r_list = jnp.asarray(r_np)
    return (r_list, out[0, :_NBINS])
